# baseline (speedup 1.0000x reference)
import functools
import math

import jax
import jax.numpy as jnp
from jax.experimental import pallas as pl
from jax.experimental.pallas import tpu as pltpu

D_X = 4096
H = 1024
NL = 64

_LOG_2PI = math.log(2.0 * math.pi)

_N_CHUNKS = 8
_CHUNK = D_X // _N_CHUNKS


def _fused_kernel(x_ref, b1_ref, b2_ref, b3_ref,
                  v1_ref, c1_ref, v2_ref, c2_ref,
                  g_ref, p1_ref, p2_ref, z1_ref, z2_ref,
                  w1_hbm, w2_hbm, w3_hbm,
                  e1_ref, e2_ref, inter_ref, logq_ref,
                  w1_v, w2_v, w3_v, *sems):
    w1_sems = sems[:_N_CHUNKS]
    w2_sem, w3_sem = sems[_N_CHUNKS], sems[_N_CHUNKS + 1]

    w1_copies = []
    for k in range(_N_CHUNKS):
        c = pltpu.make_async_copy(
            w1_hbm.at[pl.ds(k * _CHUNK, _CHUNK), :],
            w1_v.at[pl.ds(k * _CHUNK, _CHUNK), :],
            w1_sems[k])
        c.start()
        w1_copies.append(c)
    w2_copy = pltpu.make_async_copy(w2_hbm, w2_v, w2_sem)
    w2_copy.start()
    w3_copy = pltpu.make_async_copy(w3_hbm, w3_v, w3_sem)
    w3_copy.start()

    x = x_ref[...]
    acc = jnp.zeros((8, H), jnp.float32)
    for k in range(_N_CHUNKS):
        w1_copies[k].wait()
        acc = acc + jnp.dot(x[:, k * _CHUNK:(k + 1) * _CHUNK],
                            w1_v[pl.ds(k * _CHUNK, _CHUNK), :],
                            preferred_element_type=jnp.float32)
    h = jax.nn.relu(acc + b1_ref[...])

    w2_copy.wait()
    h = jax.nn.relu(jnp.dot(h, w2_v[...],
                            preferred_element_type=jnp.float32) + b2_ref[...])
    w3_copy.wait()
    o = jnp.dot(h, w3_v[...], preferred_element_type=jnp.float32) + b3_ref[...]

    e1_mean = o[0:1, 0:NL]
    e1_logstd = o[0:1, NL:2 * NL]
    e2_mean = o[1:2, 0:NL]
    e2_logstd = o[1:2, NL:2 * NL]
    e1_std = jnp.exp(e1_logstd)
    e2_std = jnp.exp(e2_logstd)

    d = jnp.abs(e1_mean - e2_mean)
    hh = jax.nn.relu(jnp.dot(d, v1_ref[...],
                             preferred_element_type=jnp.float32) + c1_ref[...])
    logits = jnp.dot(hh, v2_ref[...],
                     preferred_element_type=jnp.float32) + c2_ref[...]
    logp = jax.nn.log_softmax(logits, axis=-1)

    score = logp + g_ref[...]
    iota65 = jax.lax.broadcasted_iota(jnp.int32, (1, NL + 1), 1)
    smax = jnp.max(score)
    idx = jnp.min(jnp.where(score >= smax, iota65, NL + 1))

    onehot = (iota65 == idx).astype(jnp.float32)
    log_q_I = jnp.sum(onehot * logp)

    iota64 = jax.lax.broadcasted_iota(jnp.int32, (1, NL), 1)
    i_mask = iota64 == (idx - 1)

    p1 = p1_ref[...]
    p2 = p2_ref[...]
    eps_mean = jnp.where(i_mask, e1_mean, p1 * e1_mean + (1.0 - p1) * e2_mean)
    eps_std = jnp.where(i_mask, e1_std, p2 * e1_std + (1.0 - p2) * e2_std)

    e1 = eps_mean + jnp.sqrt(eps_std) * z1_ref[...]
    log_q_e1 = -0.5 * jnp.sum((e1 - eps_mean) ** 2 / eps_std
                              + jnp.log(eps_std) + _LOG_2PI)

    e2_samp = e2_mean + jnp.sqrt(e2_std) * z2_ref[...]
    e2 = jnp.where(i_mask, e2_samp, e1)
    per_dim = -0.5 * ((e2 - e2_mean) ** 2 / e2_std + jnp.log(e2_std) + _LOG_2PI)
    log_q_e2 = jnp.sum(jnp.where(i_mask, per_dim, 0.0))

    e1_ref[...] = e1
    e2_ref[...] = e2
    inter_ref[...] = onehot
    logq_ref[...] = jnp.full((1, 1), log_q_e1 + log_q_e2 + log_q_I,
                             dtype=jnp.float32)


@functools.partial(jax.jit, static_argnames=("interpret",))
def _run(x1, x2, W1, b1, W2, b2, W3, b3, V1, c1, V2, c2, interpret=False):
    skey = jax.random.key(1234)
    g = jax.random.gumbel(jax.random.fold_in(skey, 0), (NL + 1,), jnp.float32)
    p1 = jax.random.uniform(jax.random.fold_in(skey, 1), (NL,), jnp.float32)
    p2 = jax.random.uniform(jax.random.fold_in(skey, 2), (NL,), jnp.float32)
    z1 = jax.random.normal(jax.random.fold_in(skey, 3), (NL,), jnp.float32)
    z2 = jax.random.normal(jax.random.fold_in(skey, 4), (NL,), jnp.float32)

    X = jnp.zeros((8, D_X), jnp.float32).at[0].set(x1).at[1].set(x2)

    out_shapes = (
        jax.ShapeDtypeStruct((1, NL), jnp.float32),
        jax.ShapeDtypeStruct((1, NL), jnp.float32),
        jax.ShapeDtypeStruct((1, NL + 1), jnp.float32),
        jax.ShapeDtypeStruct((1, 1), jnp.float32),
    )
    vmem = pl.BlockSpec(memory_space=pltpu.MemorySpace.VMEM)
    hbm = pl.BlockSpec(memory_space=pltpu.MemorySpace.HBM)
    e1, e2, inter, logq = pl.pallas_call(
        _fused_kernel,
        out_shape=out_shapes,
        in_specs=[vmem] * 13 + [hbm] * 3,
        out_specs=(vmem, vmem, vmem, vmem),
        scratch_shapes=[
            pltpu.VMEM((D_X, H), jnp.float32),
            pltpu.VMEM((H, H), jnp.float32),
            pltpu.VMEM((H, 2 * NL), jnp.float32),
        ] + [pltpu.SemaphoreType.DMA] * (_N_CHUNKS + 2),
        interpret=interpret,
    )(X, b1.reshape(1, H), b2.reshape(1, H), b3.reshape(1, 2 * NL),
      V1, c1.reshape(1, 256), V2, c2.reshape(1, NL + 1),
      g.reshape(1, NL + 1), p1.reshape(1, NL), p2.reshape(1, NL),
      z1.reshape(1, NL), z2.reshape(1, NL), W1, W2, W3)
    return ((e1.reshape(NL), e2.reshape(NL), inter.reshape(NL + 1)),
            logq.reshape(()))


def kernel(x1, x2, W1, b1, W2, b2, W3, b3, V1, c1, V2, c2):
    return _run(x1, x2, W1, b1, W2, b2, W3, b3, V1, c1, V2, c2)


# speedup vs baseline: 2.5325x; 2.5325x over previous
import functools
import math

import jax
import jax.numpy as jnp
import numpy as np
from jax.experimental import pallas as pl
from jax.experimental.pallas import tpu as pltpu

D_X = 4096
H = 1024
NL = 64

_LOG_2PI = math.log(2.0 * math.pi)


def _draw_constants():
    skey = jax.random.key(1234)
    g = jax.random.gumbel(jax.random.fold_in(skey, 0), (NL + 1,), jnp.float32)
    p1 = jax.random.uniform(jax.random.fold_in(skey, 1), (NL,), jnp.float32)
    p2 = jax.random.uniform(jax.random.fold_in(skey, 2), (NL,), jnp.float32)
    z1 = jax.random.normal(jax.random.fold_in(skey, 3), (NL,), jnp.float32)
    z2 = jax.random.normal(jax.random.fold_in(skey, 4), (NL,), jnp.float32)
    return jax.tree.map(np.asarray, (g, p1, p2, z1, z2))


_G, _P1, _P2, _Z1, _Z2 = _draw_constants()

_N_CHUNKS = 8
_CHUNK = D_X // _N_CHUNKS


def _fused_kernel(x1_ref, x2_ref, b1_ref, b2_ref, b3_ref,
                  v1_ref, c1_ref, v2_ref, c2_ref,
                  g_ref, p1_ref, p2_ref, z1_ref, z2_ref,
                  w1_hbm, w2_hbm, w3_hbm,
                  e1_ref, e2_ref, inter_ref, logq_ref,
                  w1_v, w2_v, w3_v, *sems):
    w1_sems = sems[:_N_CHUNKS]
    w2_sem, w3_sem = sems[_N_CHUNKS], sems[_N_CHUNKS + 1]

    w1_copies = []
    for k in range(_N_CHUNKS):
        c = pltpu.make_async_copy(
            w1_hbm.at[pl.ds(k * _CHUNK, _CHUNK), :],
            w1_v.at[pl.ds(k * _CHUNK, _CHUNK), :],
            w1_sems[k])
        c.start()
        w1_copies.append(c)
    w2_copy = pltpu.make_async_copy(w2_hbm, w2_v, w2_sem)
    w2_copy.start()
    w3_copy = pltpu.make_async_copy(w3_hbm, w3_v, w3_sem)
    w3_copy.start()

    x = jnp.concatenate([x1_ref[...], x2_ref[...]], axis=0)
    acc = jnp.zeros((2, H), jnp.float32)
    for k in range(_N_CHUNKS):
        w1_copies[k].wait()
        acc = acc + jnp.dot(x[:, k * _CHUNK:(k + 1) * _CHUNK],
                            w1_v[pl.ds(k * _CHUNK, _CHUNK), :],
                            preferred_element_type=jnp.float32)
    h = jax.nn.relu(acc + b1_ref[...])

    w2_copy.wait()
    h = jax.nn.relu(jnp.dot(h, w2_v[...],
                            preferred_element_type=jnp.float32) + b2_ref[...])
    w3_copy.wait()
    o = jnp.dot(h, w3_v[...], preferred_element_type=jnp.float32) + b3_ref[...]

    e1_mean = o[0:1, 0:NL]
    e1_logstd = o[0:1, NL:2 * NL]
    e2_mean = o[1:2, 0:NL]
    e2_logstd = o[1:2, NL:2 * NL]
    e1_std = jnp.exp(e1_logstd)
    e2_std = jnp.exp(e2_logstd)

    d = jnp.abs(e1_mean - e2_mean)
    hh = jax.nn.relu(jnp.dot(d, v1_ref[...],
                             preferred_element_type=jnp.float32) + c1_ref[...])
    logits = jnp.dot(hh, v2_ref[...],
                     preferred_element_type=jnp.float32) + c2_ref[...]
    logp = jax.nn.log_softmax(logits, axis=-1)

    score = logp + g_ref[...]
    iota65 = jax.lax.broadcasted_iota(jnp.int32, (1, NL + 1), 1)
    smax = jnp.max(score)
    idx = jnp.min(jnp.where(score >= smax, iota65, NL + 1))

    onehot = (iota65 == idx).astype(jnp.float32)
    log_q_I = jnp.sum(onehot * logp)

    iota64 = jax.lax.broadcasted_iota(jnp.int32, (1, NL), 1)
    i_mask = iota64 == (idx - 1)

    p1 = p1_ref[...]
    p2 = p2_ref[...]
    eps_mean = jnp.where(i_mask, e1_mean, p1 * e1_mean + (1.0 - p1) * e2_mean)
    eps_std = jnp.where(i_mask, e1_std, p2 * e1_std + (1.0 - p2) * e2_std)

    e1 = eps_mean + jnp.sqrt(eps_std) * z1_ref[...]
    log_q_e1 = -0.5 * jnp.sum((e1 - eps_mean) ** 2 / eps_std
                              + jnp.log(eps_std) + _LOG_2PI)

    e2_samp = e2_mean + jnp.sqrt(e2_std) * z2_ref[...]
    e2 = jnp.where(i_mask, e2_samp, e1)
    per_dim = -0.5 * ((e2 - e2_mean) ** 2 / e2_std + jnp.log(e2_std) + _LOG_2PI)
    log_q_e2 = jnp.sum(jnp.where(i_mask, per_dim, 0.0))

    e1_ref[...] = e1
    e2_ref[...] = e2
    inter_ref[...] = onehot
    logq_ref[...] = jnp.full((1, 1), log_q_e1 + log_q_e2 + log_q_I,
                             dtype=jnp.float32)


@functools.partial(jax.jit, static_argnames=("interpret",))
def _run(x1, x2, W1, b1, W2, b2, W3, b3, V1, c1, V2, c2, interpret=False):
    g, p1, p2, z1, z2 = (jnp.asarray(a) for a in (_G, _P1, _P2, _Z1, _Z2))

    out_shapes = (
        jax.ShapeDtypeStruct((1, NL), jnp.float32),
        jax.ShapeDtypeStruct((1, NL), jnp.float32),
        jax.ShapeDtypeStruct((1, NL + 1), jnp.float32),
        jax.ShapeDtypeStruct((1, 1), jnp.float32),
    )
    vmem = pl.BlockSpec(memory_space=pltpu.MemorySpace.VMEM)
    hbm = pl.BlockSpec(memory_space=pltpu.MemorySpace.HBM)
    e1, e2, inter, logq = pl.pallas_call(
        _fused_kernel,
        out_shape=out_shapes,
        in_specs=[vmem] * 14 + [hbm] * 3,
        out_specs=(vmem, vmem, vmem, vmem),
        scratch_shapes=[
            pltpu.VMEM((D_X, H), jnp.float32),
            pltpu.VMEM((H, H), jnp.float32),
            pltpu.VMEM((H, 2 * NL), jnp.float32),
        ] + [pltpu.SemaphoreType.DMA] * (_N_CHUNKS + 2),
        interpret=interpret,
    )(x1.reshape(1, D_X), x2.reshape(1, D_X),
      b1.reshape(1, H), b2.reshape(1, H), b3.reshape(1, 2 * NL),
      V1, c1.reshape(1, 256), V2, c2.reshape(1, NL + 1),
      g.reshape(1, NL + 1), p1.reshape(1, NL), p2.reshape(1, NL),
      z1.reshape(1, NL), z2.reshape(1, NL), W1, W2, W3)
    return ((e1.reshape(NL), e2.reshape(NL), inter.reshape(NL + 1)),
            logq.reshape(()))


def kernel(x1, x2, W1, b1, W2, b2, W3, b3, V1, c1, V2, c2):
    return _run(x1, x2, W1, b1, W2, b2, W3, b3, V1, c1, V2, c2)
